# baseline (speedup 1.0000x reference)
import jax
import jax.numpy as jnp
from jax import lax
from jax.experimental import pallas as pl
from jax.experimental.pallas import tpu as pltpu
from jax.experimental.pallas import tpu_sc as plsc

_ROWS = 8192
_COLS = 1024
_NC = 2
_RPC = _ROWS // _NC
_BLK = 256
_NBLK = _RPC // _BLK
_NBUF = 4


def _scs_copy_body(w_hbm, o_hbm, *scratch):
    bufs = scratch[:_NBUF]
    sin = scratch[_NBUF:2 * _NBUF]
    sout = scratch[2 * _NBUF:3 * _NBUF]
    base = lax.axis_index("c") * _RPC

    def in_copy(i, b):
        return pltpu.make_async_copy(
            w_hbm.at[pl.ds(base + i * _BLK, _BLK), :], bufs[b], sin[b])

    def out_copy(i, b):
        return pltpu.make_async_copy(
            bufs[b], o_hbm.at[pl.ds(base + i * _BLK, _BLK), :], sout[b])

    for i in range(min(_NBUF, _NBLK)):
        in_copy(i, i % _NBUF).start()
    for i in range(_NBLK):
        b = i % _NBUF
        in_copy(i, b).wait()
        out_copy(i, b).start()
        nxt = i + _NBUF
        if nxt < _NBLK:
            out_copy(i, b).wait()
            in_copy(nxt, b).start()
    for i in range(max(0, _NBLK - _NBUF), _NBLK):
        out_copy(i, i % _NBUF).wait()


def kernel(x, W):
    del x
    mesh = plsc.ScalarSubcoreMesh(axis_name="c", num_cores=_NC)
    scratch = (
        [pltpu.VMEM_SHARED((_BLK, _COLS), jnp.float32)] * _NBUF
        + [pltpu.SemaphoreType.DMA] * (2 * _NBUF)
    )
    f = pl.kernel(
        _scs_copy_body,
        out_type=jax.ShapeDtypeStruct((_ROWS, _COLS), W.dtype),
        mesh=mesh,
        scratch_types=scratch,
    )
    return f(W)


# speedup vs baseline: 1.0711x; 1.0711x over previous
import jax
import jax.numpy as jnp
from jax import lax
from jax.experimental import pallas as pl
from jax.experimental.pallas import tpu as pltpu
from jax.experimental.pallas import tpu_sc as plsc

_ROWS = 8192
_COLS = 1024
_NC = 2
_NS = 16
_NW = _NC * _NS
_ROWS_PER_W = _ROWS // _NW
_BLK = 32
_NBLK = _ROWS_PER_W // _BLK
_NBUF = 3


def _sc_copy_body(w_hbm, o_hbm, *scratch):
    bufs = scratch[:_NBUF]
    sin = scratch[_NBUF:2 * _NBUF]
    sout = scratch[2 * _NBUF:3 * _NBUF]
    wid = lax.axis_index("s") * _NC + lax.axis_index("c")
    base = wid * _ROWS_PER_W

    def in_copy(i, b):
        return pltpu.make_async_copy(
            w_hbm.at[pl.ds(base + i * _BLK, _BLK), :], bufs[b], sin[b])

    def out_copy(i, b):
        return pltpu.make_async_copy(
            bufs[b], o_hbm.at[pl.ds(base + i * _BLK, _BLK), :], sout[b])

    for i in range(min(_NBUF, _NBLK)):
        in_copy(i, i % _NBUF).start()
    out_waited = set()
    for i in range(_NBLK):
        b = i % _NBUF
        in_copy(i, b).wait()
        out_copy(i, b).start()
        j = i - 1
        if j >= 0 and j + _NBUF < _NBLK:
            out_copy(j, j % _NBUF).wait()
            out_waited.add(j)
            in_copy(j + _NBUF, j % _NBUF).start()
    for i in range(_NBLK):
        if i not in out_waited:
            out_copy(i, i % _NBUF).wait()


def kernel(x, W):
    del x
    mesh = plsc.VectorSubcoreMesh(core_axis_name="c", subcore_axis_name="s")
    scratch = (
        [pltpu.VMEM((_BLK, _COLS), jnp.float32)] * _NBUF
        + [pltpu.SemaphoreType.DMA] * (2 * _NBUF)
    )
    f = pl.kernel(
        _sc_copy_body,
        out_type=jax.ShapeDtypeStruct((_ROWS, _COLS), W.dtype),
        mesh=mesh,
        scratch_types=scratch,
    )
    return f(W)


# speedup vs baseline: 1.1202x; 1.0458x over previous
import jax
import jax.numpy as jnp
from jax import lax
from jax.experimental import pallas as pl
from jax.experimental.pallas import tpu as pltpu
from jax.experimental.pallas import tpu_sc as plsc

_ROWS = 8192
_COLS = 1024
_NC = 2
_NS = 16
_NW = _NC * _NS
_ROWS_PER_W = _ROWS // _NW
_BLK = 16
_NBLK = _ROWS_PER_W // _BLK
_NBUF = 7
_DEFER = 2


def _sc_copy_body(w_hbm, o_hbm, *scratch):
    bufs = scratch[:_NBUF]
    sin = scratch[_NBUF:2 * _NBUF]
    sout = scratch[2 * _NBUF:3 * _NBUF]
    wid = lax.axis_index("s") * _NC + lax.axis_index("c")
    base = wid * _ROWS_PER_W

    def in_copy(i, b):
        return pltpu.make_async_copy(
            w_hbm.at[pl.ds(base + i * _BLK, _BLK), :], bufs[b], sin[b])

    def out_copy(i, b):
        return pltpu.make_async_copy(
            bufs[b], o_hbm.at[pl.ds(base + i * _BLK, _BLK), :], sout[b])

    for i in range(min(_NBUF, _NBLK)):
        in_copy(i, i % _NBUF).start()
    out_waited = set()
    for i in range(_NBLK):
        b = i % _NBUF
        in_copy(i, b).wait()
        out_copy(i, b).start()
        j = i - _DEFER
        if j >= 0 and j + _NBUF < _NBLK:
            out_copy(j, j % _NBUF).wait()
            out_waited.add(j)
            in_copy(j + _NBUF, j % _NBUF).start()
    for i in range(_NBLK):
        if i not in out_waited:
            out_copy(i, i % _NBUF).wait()


def kernel(x, W):
    del x
    mesh = plsc.VectorSubcoreMesh(core_axis_name="c", subcore_axis_name="s")
    scratch = (
        [pltpu.VMEM((_BLK, _COLS), jnp.float32)] * _NBUF
        + [pltpu.SemaphoreType.DMA] * (2 * _NBUF)
    )
    f = pl.kernel(
        _sc_copy_body,
        out_type=jax.ShapeDtypeStruct((_ROWS, _COLS), W.dtype),
        mesh=mesh,
        scratch_types=scratch,
    )
    return f(W)


# speedup vs baseline: 1.1210x; 1.0008x over previous
import jax
import jax.numpy as jnp
from jax import lax
from jax.experimental import pallas as pl
from jax.experimental.pallas import tpu as pltpu
from jax.experimental.pallas import tpu_sc as plsc

_ROWS = 8192
_COLS = 1024
_NC = 2
_NS = 16
_NW = _NC * _NS
_ROWS_PER_W = _ROWS // _NW
_BLK = 32
_NBLK = _ROWS_PER_W // _BLK
_NBUF = 3


def _sc_copy_body(w_hbm, o_hbm, *scratch):
    bufs = scratch[:_NBUF]
    sin = scratch[_NBUF:2 * _NBUF]
    sout = scratch[2 * _NBUF:3 * _NBUF]
    wid = lax.axis_index("s") * _NC + lax.axis_index("c")
    base = wid * _ROWS_PER_W

    def in_copy(i, b):
        return pltpu.make_async_copy(
            w_hbm.at[pl.ds(base + i * _BLK, _BLK), :], bufs[b], sin[b])

    def out_copy(i, b):
        return pltpu.make_async_copy(
            bufs[b], o_hbm.at[pl.ds(base + i * _BLK, _BLK), :], sout[b])

    for i in range(min(_NBUF, _NBLK)):
        in_copy(i, i % _NBUF).start()
    for i in range(_NBLK):
        b = i % _NBUF
        in_copy(i, b).wait()
        out_copy(i, b).start()
        nxt = i + _NBUF
        if nxt < _NBLK:
            out_copy(i, b).wait()
            in_copy(nxt, b).start()
    for i in range(max(0, _NBLK - _NBUF), _NBLK):
        out_copy(i, i % _NBUF).wait()


def kernel(x, W):
    del x
    mesh = plsc.VectorSubcoreMesh(core_axis_name="c", subcore_axis_name="s")
    scratch = (
        [pltpu.VMEM((_BLK, _COLS), jnp.float32)] * _NBUF
        + [pltpu.SemaphoreType.DMA] * (2 * _NBUF)
    )
    f = pl.kernel(
        _sc_copy_body,
        out_type=jax.ShapeDtypeStruct((_ROWS, _COLS), W.dtype),
        mesh=mesh,
        scratch_types=scratch,
    )
    return f(W)


# speedup vs baseline: 1.1269x; 1.0052x over previous
import jax
import jax.numpy as jnp
from jax import lax
from jax.experimental import pallas as pl
from jax.experimental.pallas import tpu as pltpu
from jax.experimental.pallas import tpu_sc as plsc

_ROWS = 8192
_COLS = 1024
_NC = 2
_NS = 16
_NW = _NC * _NS
_ROWS_PER_W = _ROWS // _NW
_BLK = 32
_NBLK = _ROWS_PER_W // _BLK
_NBUF = 3


def _sc_copy_body(w_hbm, o_hbm, *scratch):
    bufs = scratch[:_NBUF]
    sin = scratch[_NBUF:2 * _NBUF]
    wid = lax.axis_index("s") * _NC + lax.axis_index("c")
    base = wid * _ROWS_PER_W

    def in_copy(i, b):
        return pltpu.make_async_copy(
            w_hbm.at[pl.ds(base + i * _BLK, _BLK), :], bufs[b], sin[b])

    def out_copy(i, b):
        return pltpu.sync_copy(
            bufs[b], o_hbm.at[pl.ds(base + i * _BLK, _BLK), :])

    for i in range(min(_NBUF, _NBLK)):
        in_copy(i, i % _NBUF).start()
    for i in range(_NBLK):
        b = i % _NBUF
        in_copy(i, b).wait()
        out_copy(i, b)
        nxt = i + _NBUF
        if nxt < _NBLK:
            in_copy(nxt, b).start()


def kernel(x, W):
    del x
    mesh = plsc.VectorSubcoreMesh(core_axis_name="c", subcore_axis_name="s")
    scratch = (
        [pltpu.VMEM((_BLK, _COLS), jnp.float32)] * _NBUF
        + [pltpu.SemaphoreType.DMA] * _NBUF
    )
    f = pl.kernel(
        _sc_copy_body,
        out_type=jax.ShapeDtypeStruct((_ROWS, _COLS), W.dtype),
        mesh=mesh,
        scratch_types=scratch,
    )
    return f(W)
